# Initial kernel scaffold; baseline (speedup 1.0000x reference)
import jax
import jax.numpy as jnp
from jax.experimental import pallas as pl


def kernel(x, w1k, w2k, w3k, fc_wt, fc_b):
    raise NotImplementedError("write your pallas kernel here")


# speedup vs baseline: 1.4997x; 1.4997x over previous
import math

import jax
import jax.numpy as jnp
import numpy as np
from jax.experimental import pallas as pl
from jax.experimental.pallas import tpu as pltpu

_PI = math.pi

_N1 = 171
_FRAME = 96
_FQ = 24
_F2 = 6
_NCLS = 20

_ACOS_COEFS = (1.5707963050, -0.2145988016, 0.0889789874, -0.0501743046,
               0.0308918810, -0.0170881256, 0.0066700901, -0.0012624911)


def _arccos(z):
    za = jnp.minimum(jnp.abs(z), 1.0)
    poly = jnp.full_like(za, _ACOS_COEFS[7])
    for coef in reversed(_ACOS_COEFS[:7]):
        poly = poly * za + coef
    r = jnp.sqrt(jnp.maximum(1.0 - za, 0.0)) * poly
    return jnp.where(z >= 0.0, r, _PI - r)


def _angle(m):
    tr = m[0] + m[4] + m[8]
    near_pi = jnp.abs(tr + 1.0) <= 1e-4
    near_id = jnp.abs(tr - 3.0) <= 1e-4
    is_pi = jnp.logical_and(near_pi, jnp.logical_not(near_id))
    is_ac = jnp.logical_and(jnp.logical_not(near_pi), jnp.logical_not(near_id))
    ang = jnp.where(is_ac, _arccos((tr - 1.0) * 0.5),
                    jnp.where(is_pi, tr * _PI, 0.0))
    return ang, tr, is_pi, is_ac


def _rot9(w, h):
    out = []
    for i in range(3):
        for j in range(3):
            out.append(w[i * 3] * h[j] + w[i * 3 + 1] * h[3 + j]
                       + w[i * 3 + 2] * h[6 + j])
    return out


def _eps_relu(v):
    pos = jnp.where(v < 0.3, 0.3, v) * (v > 0.0).astype(v.dtype)
    neg = jnp.where(v > -0.3, -0.3, v) * (v < 0.0).astype(v.dtype)
    return pos + neg


def _compact_pool(y9, sel_ref, ngroups):
    ang, _, _, _ = _angle(y9)

    def pick(g):
        s = sel_ref[g]
        return ([jnp.dot(y9[e], s, preferred_element_type=jnp.float32)
                 for e in range(9)],
                jnp.dot(ang, s, preferred_element_type=jnp.float32))

    best, best_a = pick(0)
    for g in range(1, ngroups):
        cand, cand_a = pick(g)
        take = cand_a > best_a
        best = [jnp.where(take, cand[e], best[e]) for e in range(9)]
        best_a = jnp.where(take, cand_a, best_a)
    return best


def _body(x_ref, w1_ref, w2_ref, w3_ref, sq_ref, sr_ref, o_ref):
    halves = []
    for p in range(2):
        xs = [x_ref[0, k // 3, k % 3, pl.ds(p, _N1, 2), :] for k in range(9)]
        w = [w1_ref[e, p] for e in range(9)]
        halves.append(_rot9(w, xs))
    a0, _, _, _ = _angle(halves[0])
    a1, _, _, _ = _angle(halves[1])
    take = a1 > a0
    h1 = [jnp.where(take, halves[1][e], halves[0][e]) for e in range(9)]

    y2 = _rot9([w2_ref[e] for e in range(9)], h1)
    h2 = _compact_pool(y2, sq_ref, 4)

    y3 = _rot9([w3_ref[e] for e in range(9)], h2)
    h3 = _compact_pool(y3, sr_ref, 4)

    ang, tr, is_pi, is_ac = _angle(h3)
    c = (tr - 1.0) * 0.5
    u = (tr + 1.0) * _PI
    sin_a = jnp.where(is_ac, jnp.sqrt(jnp.maximum(1.0 - c * c, 0.0)),
                      jnp.where(is_pi, u * u * u / 6.0 - u, 0.0))
    sin_a = sin_a + (sin_a <= 1e-12).astype(sin_a.dtype)
    coef = ang / (2.0 * sin_a)
    vx = coef * (h3[7] - h3[5])
    vy = coef * (h3[6] - h3[2])
    vz = coef * (h3[3] - h3[1])
    nrm = jnp.sqrt(vx * vx + vy * vy + vz * vz)
    comps = [_eps_relu(v) for v in (vx / nrm, vy / nrm, vz / nrm, ang)]
    o_ref[0] = jnp.concatenate(comps, axis=1)


def _head(f_ref, w_ref, b_ref, o_ref):
    logits = jnp.dot(f_ref[...], w_ref[...],
                     preferred_element_type=jnp.float32) + b_ref[...]
    z = jnp.exp(logits - jnp.max(logits, axis=1, keepdims=True))
    o_ref[...] = z / jnp.sum(z, axis=1, keepdims=True)


def _onehot_selectors():
    sq = np.zeros((4, _FRAME, _FQ), np.float32)
    for q in range(4):
        for t in range(_FQ):
            sq[q, (t // 4) * 16 + (t % 4) * 4 + q, t] = 1.0
    sr = np.zeros((4, _FQ, _F2), np.float32)
    for r in range(4):
        for u in range(_F2):
            sr[r, u * 4 + r, u] = 1.0
    return jnp.asarray(sq), jnp.asarray(sr)


def kernel(x, w1k, w2k, w3k, fc_wt, fc_b):
    n = x.shape[0]
    w1b = jnp.broadcast_to(w1k.reshape(9, 2, _N1)[..., None],
                           (9, 2, _N1, _FRAME))
    w2b = jnp.broadcast_to(w2k.reshape(9, _N1)[..., None], (9, _N1, _FRAME))
    w3b = jnp.broadcast_to(w3k.reshape(9, _N1)[..., None], (9, _N1, _FQ))
    sq, sr = _onehot_selectors()

    feats = pl.pallas_call(
        _body,
        out_shape=jax.ShapeDtypeStruct((n, _N1, _FQ), jnp.float32),
        grid=(n,),
        in_specs=[
            pl.BlockSpec((1, 3, 3, 2 * _N1, _FRAME), lambda i: (i, 0, 0, 0, 0)),
            pl.BlockSpec((9, 2, _N1, _FRAME), lambda i: (0, 0, 0, 0)),
            pl.BlockSpec((9, _N1, _FRAME), lambda i: (0, 0, 0)),
            pl.BlockSpec((9, _N1, _FQ), lambda i: (0, 0, 0)),
            pl.BlockSpec((4, _FRAME, _FQ), lambda i: (0, 0, 0)),
            pl.BlockSpec((4, _FQ, _F2), lambda i: (0, 0, 0)),
        ],
        out_specs=pl.BlockSpec((1, _N1, _FQ), lambda i: (i, 0, 0)),
        compiler_params=pltpu.CompilerParams(
            dimension_semantics=("parallel",),
            vmem_limit_bytes=40 * 1024 * 1024,
        ),
    )(x, w1b, w2b, w3b, sq, sr)

    wct = fc_wt.reshape(4, _F2, _N1, _NCLS).transpose(2, 0, 1, 3)
    wct = wct.reshape(4 * _F2 * _N1, _NCLS)
    return pl.pallas_call(
        _head,
        out_shape=jax.ShapeDtypeStruct((n, _NCLS), jnp.float32),
    )(feats.reshape(n, 4 * _F2 * _N1), wct, fc_b)


# speedup vs baseline: 1.6556x; 1.1040x over previous
import math

import jax
import jax.numpy as jnp
import numpy as np
from jax.experimental import pallas as pl
from jax.experimental.pallas import tpu as pltpu

_PI = math.pi

_N1 = 171
_FRAME = 96
_FQ = 24
_F2 = 6
_NCLS = 20

_ACOS_COEFS = (1.5707963050, -0.2145988016, 0.0889789874, -0.0501743046,
               0.0308918810, -0.0170881256, 0.0066700901, -0.0012624911)


def _arccos(z):
    za = jnp.minimum(jnp.abs(z), 1.0)
    poly = jnp.full_like(za, _ACOS_COEFS[7])
    for coef in reversed(_ACOS_COEFS[:7]):
        poly = poly * za + coef
    r = jnp.sqrt(jnp.maximum(1.0 - za, 0.0)) * poly
    return jnp.where(z >= 0.0, r, _PI - r)


def _angle(m):
    tr = m[0] + m[4] + m[8]
    near_pi = jnp.abs(tr + 1.0) <= 1e-4
    near_id = jnp.abs(tr - 3.0) <= 1e-4
    is_pi = jnp.logical_and(near_pi, jnp.logical_not(near_id))
    is_ac = jnp.logical_and(jnp.logical_not(near_pi), jnp.logical_not(near_id))
    ang = jnp.where(is_ac, _arccos((tr - 1.0) * 0.5),
                    jnp.where(is_pi, tr * _PI, 0.0))
    return ang, tr, is_pi, is_ac


def _pool_key(m):
    tr = m[0] + m[4] + m[8]
    return jnp.where(jnp.abs(tr + 1.0) <= 1e-4, tr - 16.0, -tr)


def _rot9(w, h):
    out = []
    for i in range(3):
        for j in range(3):
            out.append(w[i * 3] * h[j] + w[i * 3 + 1] * h[3 + j]
                       + w[i * 3 + 2] * h[6 + j])
    return out


def _eps_relu(v):
    pos = jnp.where(v < 0.3, 0.3, v) * (v > 0.0).astype(v.dtype)
    neg = jnp.where(v > -0.3, -0.3, v) * (v < 0.0).astype(v.dtype)
    return pos + neg


def _compact_pool(y9, sel_ref, ngroups):
    key = _pool_key(y9)

    def pick(g):
        s = sel_ref[g]
        return ([jnp.dot(y9[e], s, preferred_element_type=jnp.float32)
                 for e in range(9)],
                jnp.dot(key, s, preferred_element_type=jnp.float32))

    best, best_a = pick(0)
    for g in range(1, ngroups):
        cand, cand_a = pick(g)
        take = cand_a > best_a
        best = [jnp.where(take, cand[e], best[e]) for e in range(9)]
        best_a = jnp.where(take, cand_a, best_a)
    return best


def _body(x_ref, w1_ref, w2_ref, w3_ref, sq_ref, sr_ref, o_ref):
    halves = []
    for p in range(2):
        xs = [x_ref[0, k // 3, k % 3, pl.ds(p, _N1, 2), :] for k in range(9)]
        w = [w1_ref[e, p] for e in range(9)]
        halves.append(_rot9(w, xs))
    take = _pool_key(halves[1]) > _pool_key(halves[0])
    h1 = [jnp.where(take, halves[1][e], halves[0][e]) for e in range(9)]

    y2 = _rot9([w2_ref[e] for e in range(9)], h1)
    h2 = _compact_pool(y2, sq_ref, 4)

    y3 = _rot9([w3_ref[e] for e in range(9)], h2)
    h3 = _compact_pool(y3, sr_ref, 4)

    ang, tr, is_pi, is_ac = _angle(h3)
    c = (tr - 1.0) * 0.5
    u = (tr + 1.0) * _PI
    sin_a = jnp.where(is_ac, jnp.sqrt(jnp.maximum(1.0 - c * c, 0.0)),
                      jnp.where(is_pi, u * u * u / 6.0 - u, 0.0))
    sin_a = sin_a + (sin_a <= 1e-12).astype(sin_a.dtype)
    coef = ang / (2.0 * sin_a)
    vx = coef * (h3[7] - h3[5])
    vy = coef * (h3[6] - h3[2])
    vz = coef * (h3[3] - h3[1])
    nrm = jnp.sqrt(vx * vx + vy * vy + vz * vz)
    comps = [_eps_relu(v) for v in (vx / nrm, vy / nrm, vz / nrm, ang)]
    o_ref[0] = jnp.concatenate(comps, axis=1)


def _head(f_ref, w_ref, b_ref, o_ref):
    logits = jnp.dot(f_ref[...], w_ref[...],
                     preferred_element_type=jnp.float32) + b_ref[...]
    z = jnp.exp(logits - jnp.max(logits, axis=1, keepdims=True))
    o_ref[...] = z / jnp.sum(z, axis=1, keepdims=True)


def _onehot_selectors():
    sq = np.zeros((4, _FRAME, _FQ), np.float32)
    for q in range(4):
        for t in range(_FQ):
            sq[q, (t // 4) * 16 + (t % 4) * 4 + q, t] = 1.0
    sr = np.zeros((4, _FQ, _F2), np.float32)
    for r in range(4):
        for u in range(_F2):
            sr[r, u * 4 + r, u] = 1.0
    return jnp.asarray(sq), jnp.asarray(sr)


def kernel(x, w1k, w2k, w3k, fc_wt, fc_b):
    n = x.shape[0]
    w1b = jnp.broadcast_to(w1k.reshape(9, 2, _N1)[..., None],
                           (9, 2, _N1, _FRAME))
    w2b = jnp.broadcast_to(w2k.reshape(9, _N1)[..., None], (9, _N1, _FRAME))
    w3b = jnp.broadcast_to(w3k.reshape(9, _N1)[..., None], (9, _N1, _FQ))
    sq, sr = _onehot_selectors()

    feats = pl.pallas_call(
        _body,
        out_shape=jax.ShapeDtypeStruct((n, _N1, _FQ), jnp.float32),
        grid=(n,),
        in_specs=[
            pl.BlockSpec((1, 3, 3, 2 * _N1, _FRAME), lambda i: (i, 0, 0, 0, 0)),
            pl.BlockSpec((9, 2, _N1, _FRAME), lambda i: (0, 0, 0, 0)),
            pl.BlockSpec((9, _N1, _FRAME), lambda i: (0, 0, 0)),
            pl.BlockSpec((9, _N1, _FQ), lambda i: (0, 0, 0)),
            pl.BlockSpec((4, _FRAME, _FQ), lambda i: (0, 0, 0)),
            pl.BlockSpec((4, _FQ, _F2), lambda i: (0, 0, 0)),
        ],
        out_specs=pl.BlockSpec((1, _N1, _FQ), lambda i: (i, 0, 0)),
        compiler_params=pltpu.CompilerParams(
            dimension_semantics=("parallel",),
            vmem_limit_bytes=40 * 1024 * 1024,
        ),
    )(x, w1b, w2b, w3b, sq, sr)

    wct = fc_wt.reshape(4, _F2, _N1, _NCLS).transpose(2, 0, 1, 3)
    wct = wct.reshape(4 * _F2 * _N1, _NCLS)
    return pl.pallas_call(
        _head,
        out_shape=jax.ShapeDtypeStruct((n, _NCLS), jnp.float32),
    )(feats.reshape(n, 4 * _F2 * _N1), wct, fc_b)


# speedup vs baseline: 1.8570x; 1.1216x over previous
import functools
import math

import jax
import jax.numpy as jnp
import numpy as np
from jax.experimental import pallas as pl
from jax.experimental.pallas import tpu as pltpu

_PI = math.pi

_N1 = 171
_FRAME = 96
_FQ = 24
_F2 = 6
_NCLS = 20
_G = 4

_ACOS_COEFS = (1.5707963050, -0.2145988016, 0.0889789874, -0.0501743046,
               0.0308918810, -0.0170881256, 0.0066700901, -0.0012624911)


def _arccos(z):
    za = jnp.minimum(jnp.abs(z), 1.0)
    poly = jnp.full_like(za, _ACOS_COEFS[7])
    for coef in reversed(_ACOS_COEFS[:7]):
        poly = poly * za + coef
    r = jnp.sqrt(jnp.maximum(1.0 - za, 0.0)) * poly
    return jnp.where(z >= 0.0, r, _PI - r)


def _angle(m):
    tr = m[0] + m[4] + m[8]
    near_pi = jnp.abs(tr + 1.0) <= 1e-4
    near_id = jnp.abs(tr - 3.0) <= 1e-4
    is_pi = jnp.logical_and(near_pi, jnp.logical_not(near_id))
    is_ac = jnp.logical_and(jnp.logical_not(near_pi), jnp.logical_not(near_id))
    ang = jnp.where(is_ac, _arccos((tr - 1.0) * 0.5),
                    jnp.where(is_pi, tr * _PI, 0.0))
    return ang, tr, is_pi, is_ac


def _pool_key(m):
    tr = m[0] + m[4] + m[8]
    return jnp.where(jnp.abs(tr + 1.0) <= 1e-4, tr - 16.0, -tr)


def _rot9(w, h):
    out = []
    for i in range(3):
        for j in range(3):
            out.append(w[i * 3] * h[j] + w[i * 3 + 1] * h[3 + j]
                       + w[i * 3 + 2] * h[6 + j])
    return out


def _eps_relu(v):
    pos = jnp.where(v < 0.3, 0.3, v) * (v > 0.0).astype(v.dtype)
    neg = jnp.where(v > -0.3, -0.3, v) * (v < 0.0).astype(v.dtype)
    return pos + neg


def _select_max(cands):
    best, best_k = cands[0]
    for cand, cand_k in cands[1:]:
        take = cand_k > best_k
        best = [jnp.where(take, cand[e], best[e]) for e in range(9)]
        best_k = jnp.where(take, cand_k, best_k)
    return best


def _body(x_ref, w1_ref, w2_ref, w3_ref, sq_ref, sr_ref, o_ref):
    g = x_ref.shape[0]
    per_sample = []
    for s in range(g):
        halves = []
        for p in range(2):
            xs = [x_ref[s, k // 3, k % 3, pl.ds(p, _N1, 2), :]
                  for k in range(9)]
            halves.append(_rot9([w1_ref[e, p] for e in range(9)], xs))
        take = _pool_key(halves[1]) > _pool_key(halves[0])
        h1 = [jnp.where(take, halves[1][e], halves[0][e]) for e in range(9)]

        y2 = _rot9([w2_ref[e] for e in range(9)], h1)
        per_sample.append((y2, _pool_key(y2)))

    def q_cand(q):
        slabs = [functools.reduce(
            jnp.add,
            [jnp.dot(per_sample[s][0][e], sq_ref[s, q],
                     preferred_element_type=jnp.float32) for s in range(g)])
            for e in range(9)]
        key = functools.reduce(
            jnp.add,
            [jnp.dot(per_sample[s][1], sq_ref[s, q],
                     preferred_element_type=jnp.float32) for s in range(g)])
        return slabs, key

    h2 = _select_max([q_cand(q) for q in range(4)])

    y3 = _rot9([w3_ref[e] for e in range(9)], h2)
    k3 = _pool_key(y3)

    def r_cand(r):
        return ([jnp.dot(y3[e], sr_ref[r], preferred_element_type=jnp.float32)
                 for e in range(9)],
                jnp.dot(k3, sr_ref[r], preferred_element_type=jnp.float32))

    h3 = _select_max([r_cand(r) for r in range(4)])

    ang, tr, is_pi, is_ac = _angle(h3)
    c = (tr - 1.0) * 0.5
    u = (tr + 1.0) * _PI
    sin_a = jnp.where(is_ac, jnp.sqrt(jnp.maximum(1.0 - c * c, 0.0)),
                      jnp.where(is_pi, u * u * u / 6.0 - u, 0.0))
    sin_a = sin_a + (sin_a <= 1e-12).astype(sin_a.dtype)
    coef = ang / (2.0 * sin_a)
    vx = coef * (h3[7] - h3[5])
    vy = coef * (h3[6] - h3[2])
    vz = coef * (h3[3] - h3[1])
    nrm = jnp.sqrt(vx * vx + vy * vy + vz * vz)
    comps = [_eps_relu(v) for v in (vx / nrm, vy / nrm, vz / nrm, ang)]
    o_ref[0] = jnp.concatenate(comps, axis=1)


def _head(f_ref, w_ref, b_ref, o_ref):
    logits = jnp.dot(f_ref[...], w_ref[...],
                     preferred_element_type=jnp.float32) + b_ref[...]
    z = jnp.exp(logits - jnp.max(logits, axis=1, keepdims=True))
    o_ref[...] = z / jnp.sum(z, axis=1, keepdims=True)


def _onehot_selectors(g):
    sq = np.zeros((g, 4, _FRAME, g * _FQ), np.float32)
    for s in range(g):
        for q in range(4):
            for t in range(_FQ):
                sq[s, q, (t // 4) * 16 + (t % 4) * 4 + q, s * _FQ + t] = 1.0
    sr = np.zeros((4, g * _FQ, g * _F2), np.float32)
    for r in range(4):
        for s in range(g):
            for u in range(_F2):
                sr[r, s * _FQ + u * 4 + r, s * _F2 + u] = 1.0
    return jnp.asarray(sq), jnp.asarray(sr)


def kernel(x, w1k, w2k, w3k, fc_wt, fc_b):
    n = x.shape[0]
    g = _G if n % _G == 0 else 1
    nb = n // g
    w1b = jnp.broadcast_to(w1k.reshape(9, 2, _N1)[..., None],
                           (9, 2, _N1, _FRAME))
    w2b = jnp.broadcast_to(w2k.reshape(9, _N1)[..., None], (9, _N1, _FRAME))
    w3b = jnp.broadcast_to(w3k.reshape(9, _N1)[..., None], (9, _N1, g * _FQ))
    sq, sr = _onehot_selectors(g)

    packed = pl.pallas_call(
        _body,
        out_shape=jax.ShapeDtypeStruct((nb, _N1, g * 4 * _F2), jnp.float32),
        grid=(nb,),
        in_specs=[
            pl.BlockSpec((g, 3, 3, 2 * _N1, _FRAME), lambda i: (i, 0, 0, 0, 0)),
            pl.BlockSpec((9, 2, _N1, _FRAME), lambda i: (0, 0, 0, 0)),
            pl.BlockSpec((9, _N1, _FRAME), lambda i: (0, 0, 0)),
            pl.BlockSpec((9, _N1, g * _FQ), lambda i: (0, 0, 0)),
            pl.BlockSpec((g, 4, _FRAME, g * _FQ), lambda i: (0, 0, 0, 0)),
            pl.BlockSpec((4, g * _FQ, g * _F2), lambda i: (0, 0, 0)),
        ],
        out_specs=pl.BlockSpec((1, _N1, g * 4 * _F2), lambda i: (i, 0, 0)),
        compiler_params=pltpu.CompilerParams(
            dimension_semantics=("parallel",),
            vmem_limit_bytes=48 * 1024 * 1024,
        ),
    )(x, w1b, w2b, w3b, sq, sr)

    feats = packed.reshape(nb, _N1, 4, g, _F2).transpose(0, 3, 1, 2, 4)
    feats = feats.reshape(n, 4 * _N1 * _F2)
    wct = fc_wt.reshape(4, _F2, _N1, _NCLS).transpose(2, 0, 1, 3)
    wct = wct.reshape(4 * _F2 * _N1, _NCLS)
    return pl.pallas_call(
        _head,
        out_shape=jax.ShapeDtypeStruct((n, _NCLS), jnp.float32),
    )(feats, wct, fc_b)


# speedup vs baseline: 2.2715x; 1.2232x over previous
import functools
import math

import jax
import jax.numpy as jnp
import numpy as np
from jax.experimental import pallas as pl
from jax.experimental.pallas import tpu as pltpu

_PI = math.pi

_N1 = 171
_FRAME = 96
_FQ = 24
_F2 = 6
_NCLS = 20
_G = 4

_ACOS_COEFS = (1.5707963050, -0.2145988016, 0.0889789874, -0.0501743046,
               0.0308918810, -0.0170881256, 0.0066700901, -0.0012624911)


def _arccos(z):
    za = jnp.minimum(jnp.abs(z), 1.0)
    poly = jnp.full_like(za, _ACOS_COEFS[7])
    for coef in reversed(_ACOS_COEFS[:7]):
        poly = poly * za + coef
    r = jnp.sqrt(jnp.maximum(1.0 - za, 0.0)) * poly
    return jnp.where(z >= 0.0, r, _PI - r)


def _angle(m):
    tr = m[0] + m[4] + m[8]
    near_pi = jnp.abs(tr + 1.0) <= 1e-4
    near_id = jnp.abs(tr - 3.0) <= 1e-4
    is_pi = jnp.logical_and(near_pi, jnp.logical_not(near_id))
    is_ac = jnp.logical_and(jnp.logical_not(near_pi), jnp.logical_not(near_id))
    ang = jnp.where(is_ac, _arccos((tr - 1.0) * 0.5),
                    jnp.where(is_pi, tr * _PI, 0.0))
    return ang, tr, is_pi, is_ac


def _pool_key(m):
    tr = m[0] + m[4] + m[8]
    return jnp.where(jnp.abs(tr + 1.0) <= 1e-4, tr - 16.0, -tr)


def _rot9(w, h):
    out = []
    for i in range(3):
        for j in range(3):
            out.append(w[i * 3] * h[j] + w[i * 3 + 1] * h[3 + j]
                       + w[i * 3 + 2] * h[6 + j])
    return out


def _eps_relu(v):
    pos = jnp.where(v < 0.3, 0.3, v) * (v > 0.0).astype(v.dtype)
    neg = jnp.where(v > -0.3, -0.3, v) * (v < 0.0).astype(v.dtype)
    return pos + neg


def _select_max(cands):
    best, best_k = cands[0]
    for cand, cand_k in cands[1:]:
        take = cand_k > best_k
        best = [jnp.where(take, cand[e], best[e]) for e in range(9)]
        best_k = jnp.where(take, cand_k, best_k)
    return best


def _body(x_ref, w1_ref, w2_ref, w3_ref, sq_ref, sr_ref, eye_ref, o_ref):
    g = x_ref.shape[0]
    per_sample = []
    for s in range(g):
        halves = []
        for p in range(2):
            xs = [x_ref[s, k // 3, k % 3, pl.ds(p, _N1, 2), :]
                  for k in range(9)]
            halves.append(_rot9([w1_ref[e, p] for e in range(9)], xs))
        take = _pool_key(halves[1]) > _pool_key(halves[0])
        h1 = [jnp.where(take, halves[1][e], halves[0][e]) for e in range(9)]

        y2 = _rot9([w2_ref[e] for e in range(9)], h1)
        per_sample.append((y2, _pool_key(y2)))

    def q_cand(q):
        slabs = [functools.reduce(
            jnp.add,
            [jnp.dot(per_sample[s][0][e], sq_ref[s, q],
                     preferred_element_type=jnp.float32) for s in range(g)])
            for e in range(9)]
        key = functools.reduce(
            jnp.add,
            [jnp.dot(per_sample[s][1], sq_ref[s, q],
                     preferred_element_type=jnp.float32) for s in range(g)])
        return slabs, key

    h2 = _select_max([q_cand(q) for q in range(4)])

    y3 = _rot9([w3_ref[e] for e in range(9)], h2)
    k3 = _pool_key(y3)

    def r_cand(r):
        return ([jnp.dot(y3[e], sr_ref[r], preferred_element_type=jnp.float32)
                 for e in range(9)],
                jnp.dot(k3, sr_ref[r], preferred_element_type=jnp.float32))

    h3 = _select_max([r_cand(r) for r in range(4)])

    ang, tr, is_pi, is_ac = _angle(h3)
    c = (tr - 1.0) * 0.5
    u = (tr + 1.0) * _PI
    sin_a = jnp.where(is_ac, jnp.sqrt(jnp.maximum(1.0 - c * c, 0.0)),
                      jnp.where(is_pi, u * u * u / 6.0 - u, 0.0))
    sin_a = sin_a + (sin_a <= 1e-12).astype(sin_a.dtype)
    coef = ang / (2.0 * sin_a)
    vx = coef * (h3[7] - h3[5])
    vy = coef * (h3[6] - h3[2])
    vz = coef * (h3[3] - h3[1])
    nrm = jnp.sqrt(vx * vx + vy * vy + vz * vz)
    comps = [_eps_relu(v) for v in (vx / nrm, vy / nrm, vz / nrm, ang)]
    feats = jnp.concatenate(comps, axis=1)
    o_ref[0] = jax.lax.dot_general(
        feats, eye_ref[...], (((0,), (0,)), ((), ())),
        preferred_element_type=jnp.float32)


def _head(f_ref, w_ref, b_ref, o_ref):
    logits = jnp.dot(f_ref[...], w_ref[...],
                     preferred_element_type=jnp.float32) + b_ref[...]
    z = jnp.exp(logits - jnp.max(logits, axis=1, keepdims=True))
    o_ref[...] = z / jnp.sum(z, axis=1, keepdims=True)


def _onehot_selectors(g):
    sq = np.zeros((g, 4, _FRAME, g * _FQ), np.float32)
    for s in range(g):
        for q in range(4):
            for t in range(_FQ):
                sq[s, q, (t // 4) * 16 + (t % 4) * 4 + q, s * _FQ + t] = 1.0
    sr = np.zeros((4, g * _FQ, g * _F2), np.float32)
    for r in range(4):
        for s in range(g):
            for u in range(_F2):
                sr[r, s * _FQ + u * 4 + r, s * _F2 + u] = 1.0
    return jnp.asarray(sq), jnp.asarray(sr)


def kernel(x, w1k, w2k, w3k, fc_wt, fc_b):
    n = x.shape[0]
    g = _G if n % _G == 0 else 1
    nb = n // g
    w1b = jnp.broadcast_to(w1k.reshape(9, 2, _N1)[..., None],
                           (9, 2, _N1, _FRAME))
    w2b = jnp.broadcast_to(w2k.reshape(9, _N1)[..., None], (9, _N1, _FRAME))
    w3b = jnp.broadcast_to(w3k.reshape(9, _N1)[..., None], (9, _N1, g * _FQ))
    sq, sr = _onehot_selectors(g)
    eye = jnp.eye(_N1, dtype=jnp.float32)

    packed = pl.pallas_call(
        _body,
        out_shape=jax.ShapeDtypeStruct((nb, g * 4 * _F2, _N1), jnp.float32),
        grid=(nb,),
        in_specs=[
            pl.BlockSpec((g, 3, 3, 2 * _N1, _FRAME), lambda i: (i, 0, 0, 0, 0)),
            pl.BlockSpec((9, 2, _N1, _FRAME), lambda i: (0, 0, 0, 0)),
            pl.BlockSpec((9, _N1, _FRAME), lambda i: (0, 0, 0)),
            pl.BlockSpec((9, _N1, g * _FQ), lambda i: (0, 0, 0)),
            pl.BlockSpec((g, 4, _FRAME, g * _FQ), lambda i: (0, 0, 0, 0)),
            pl.BlockSpec((4, g * _FQ, g * _F2), lambda i: (0, 0, 0)),
            pl.BlockSpec((_N1, _N1), lambda i: (0, 0)),
        ],
        out_specs=pl.BlockSpec((1, g * 4 * _F2, _N1), lambda i: (i, 0, 0)),
        compiler_params=pltpu.CompilerParams(
            dimension_semantics=("parallel",),
            vmem_limit_bytes=48 * 1024 * 1024,
        ),
    )(x, w1b, w2b, w3b, sq, sr, eye)

    feats = packed.reshape(nb, 4, g, _F2, _N1).transpose(0, 2, 1, 3, 4)
    feats = feats.reshape(n, 4 * _N1 * _F2)
    return pl.pallas_call(
        _head,
        out_shape=jax.ShapeDtypeStruct((n, _NCLS), jnp.float32),
    )(feats, fc_wt, fc_b)
